# baseline (speedup 1.0000x reference)
import functools

import jax
import jax.numpy as jnp
import numpy as np
from jax import lax
from jax.experimental import pallas as pl
from jax.experimental.pallas import tpu as pltpu
from jax.experimental.pallas import tpu_sc as plsc

_NBINS = 256
_D = 512
_B = 16 * 4096
_NW = 32
_BPW = _B // _NW
_CHUNK = 128
_NCHUNK = _BPW // _CHUNK
_TABW = (_NBINS - 1) * _D

_F0_MIN = 50.0
_F0_MAX = 1100.0


def _prep_body(f0_ref, emb_ref, w1_ref, b1_ref, w2_ref, b2_ref, idx_ref, tab_ref):
    mel_min = 1127.0 * float(np.log(1.0 + _F0_MIN / 700.0))
    mel_max = 1127.0 * float(np.log(1.0 + _F0_MAX / 700.0))
    f0 = f0_ref[...]
    mel = 1127.0 * jnp.log(1.0 + f0 / 700.0)
    mel = jnp.where(
        mel > 0.0,
        (mel - mel_min) * (_NBINS - 2) / (mel_max - mel_min) + 1.0,
        mel,
    )
    mel = jnp.where(mel <= 1.0, 1.0, mel)
    mel = jnp.where(mel > _NBINS - 1, float(_NBINS - 1), mel)
    idx_ref[...] = ((mel + 0.5).astype(jnp.int32) - 1) * _D

    h = jnp.dot(emb_ref[...], w1_ref[...], preferred_element_type=jnp.float32)
    h = h + b1_ref[...]
    h = h * 0.5 * (1.0 + lax.erf(h * np.float32(1.0 / np.sqrt(2.0))))
    tab_ref[...] = (
        jnp.dot(h, w2_ref[...], preferred_element_type=jnp.float32) + b2_ref[...]
    )


def _prep(f0v, emb, w1, b1, w2, b2):
    return pl.pallas_call(
        _prep_body,
        out_shape=(
            jax.ShapeDtypeStruct((_B // _CHUNK, _CHUNK), jnp.int32),
            jax.ShapeDtypeStruct((_NBINS, _D), jnp.float32),
        ),
    )(f0v, emb, w1, b1, w2, b2)


def _sc_emit(tab, idx2d):
    tab1d = tab.reshape(_NBINS * _D)
    mesh = plsc.VectorSubcoreMesh(core_axis_name="c", subcore_axis_name="s")

    @functools.partial(
        pl.kernel,
        mesh=mesh,
        out_type=jax.ShapeDtypeStruct((_B, _D), jnp.float32),
        scratch_types=[
            pltpu.VMEM((_TABW,), jnp.float32),
            pltpu.VMEM((_CHUNK,), jnp.int32),
            pltpu.SemaphoreType.DMA,
            pltpu.SemaphoreType.DMA,
        ],
    )
    def k(tab_hbm, idx_hbm, out_hbm, tab_v, idx_v, sem, drain_sem):
        cid = lax.axis_index("c")
        sid = lax.axis_index("s")
        wid = sid * 2 + cid
        base = wid * _BPW

        pltpu.sync_copy(tab_hbm.at[pl.ds(_D, _TABW)], tab_v)

        def fire_chunk(c):
            pltpu.sync_copy(idx_hbm.at[wid * _NCHUNK + c], idx_v)

            def fire(g, carry):
                offs = idx_v[pl.ds(g * 16, 16)]
                for i in range(16):
                    pltpu.async_copy(
                        tab_v.at[pl.ds(pl.multiple_of(offs[i], _D), _D)],
                        out_hbm.at[base + c * _CHUNK + g * 16 + i],
                        sem,
                    )
                return carry

            lax.fori_loop(0, _CHUNK // 16, fire, 0)

        def drain_chunk():
            pltpu.make_async_copy(
                out_hbm.at[pl.ds(0, _CHUNK)],
                out_hbm.at[pl.ds(0, _CHUNK)],
                sem,
            ).wait()

        fire_chunk(0)

        def body(c, carry):
            fire_chunk(c)
            drain_chunk()
            return carry

        lax.fori_loop(1, _NCHUNK, body, 0)
        drain_chunk()

    return k(tab1d, idx2d)


def kernel(f0, emb, W1, b1, W2, b2):
    f0v = f0.reshape(_B // _CHUNK, _CHUNK)
    idx2d, tab = _prep(f0v, emb, W1, b1.reshape(1, _D), W2, b2.reshape(1, _D))
    out = _sc_emit(tab, idx2d)
    return out.reshape(f0.shape[0], f0.shape[1], _D)
